# baseline (speedup 1.0000x reference)
import functools

import jax
import jax.numpy as jnp
from jax import lax
from jax.experimental import pallas as pl
from jax.experimental.pallas import tpu as pltpu
from jax.experimental.pallas import tpu_sc as plsc

BATCH = 16384
D = 64
NC, NS = 2, 16
NW = NC * NS
BPW = BATCH // NW
CHUNK = 128
NCHUNK = BPW // CHUNK
L = 16


def _gather_table(tab2, idx_v, out_hbm, base, rowbufs, sems):
    def issue(cc, b):
        def it(t, _):
            rvec = idx_v[pl.ds(cc * CHUNK + t * L, L)]
            qvec = lax.shift_right_logical(rvec, 4)
            svec = jnp.bitwise_and(lax.shift_right_logical(rvec, 1), 7)
            hvec = jnp.bitwise_and(rvec, 1)
            for lane in range(L):
                q = qvec[lane]
                s = svec[lane]
                h = hvec[lane]
                i = t * L + lane
                dst = rowbufs[b].at[i // 16, pl.ds((i // 2) % 8, 1),
                                    pl.ds((i % 2) * D, D)]

                @pl.when(h == 0)
                def _():
                    pltpu.make_async_copy(
                        tab2.at[q, pl.ds(s, 1), pl.ds(0, D)], dst, sems[b]
                    ).start()

                @pl.when(h == 1)
                def _():
                    pltpu.make_async_copy(
                        tab2.at[q, pl.ds(s, 1), pl.ds(D, D)], dst, sems[b]
                    ).start()
            return ()

        lax.fori_loop(0, CHUNK // L, it, ())

    def drain(b):
        pltpu.make_async_copy(
            out_hbm.at[pl.ds(0, CHUNK // 16)], rowbufs[b], sems[b]
        ).wait()

    issue(0, 0)
    for cc in range(NCHUNK):
        b = cc % 2
        nxt = cc + 1
        if nxt < NCHUNK:
            issue(nxt, 1 - b)
        drain(b)
        pltpu.sync_copy(
            rowbufs[b],
            out_hbm.at[pl.ds((base + cc * CHUNK) // 16, CHUNK // 16)],
        )


def _gather_body(uidx_hbm, midx_hbm, utab2_hbm, mtab2_hbm,
                 uout_hbm, mout_hbm,
                 uidx_v, midx_v, rowbuf0, rowbuf1, sem0, sem1):
    wid = lax.axis_index("s") * NC + lax.axis_index("c")
    base = wid * BPW
    pltpu.sync_copy(uidx_hbm.at[pl.ds(base, BPW)], uidx_v)
    pltpu.sync_copy(midx_hbm.at[pl.ds(base, BPW)], midx_v)
    rowbufs = (rowbuf0, rowbuf1)
    sems = (sem0, sem1)
    _gather_table(utab2_hbm, uidx_v, uout_hbm, base, rowbufs, sems)
    _gather_table(mtab2_hbm, midx_v, mout_hbm, base, rowbufs, sems)


@functools.cache
def _make_gather():
    return pl.kernel(
        _gather_body,
        out_type=(jax.ShapeDtypeStruct((BATCH // 16, 8, 2 * D), jnp.float32),
                  jax.ShapeDtypeStruct((BATCH // 16, 8, 2 * D), jnp.float32)),
        mesh=plsc.VectorSubcoreMesh(core_axis_name="c", subcore_axis_name="s",
                                    num_cores=NC, num_subcores=NS),
        compiler_params=pltpu.CompilerParams(
            needs_layout_passes=False,
            disable_bounds_checks=True,
            disable_semaphore_checks=True,
            skip_device_barrier=True,
        ),
        scratch_types=[
            pltpu.VMEM((BPW,), jnp.int32),
            pltpu.VMEM((BPW,), jnp.int32),
            pltpu.VMEM((CHUNK // 16, 8, 2 * D), jnp.float32),
            pltpu.VMEM((CHUNK // 16, 8, 2 * D), jnp.float32),
            pltpu.SemaphoreType.DMA,
            pltpu.SemaphoreType.DMA,
        ],
    )


BLK = 2048


def _mlp_body(ue_ref, me_ref, w1a_ref, w1b_ref, b1_ref, w2_ref, b2_ref,
              w3_ref, b3_ref, o_ref):
    h = jnp.dot(ue_ref[...], w1a_ref[...], preferred_element_type=jnp.float32)
    h = h + jnp.dot(me_ref[...], w1b_ref[...],
                    preferred_element_type=jnp.float32)
    h = jnp.maximum(h + b1_ref[...], 0.0)
    h = jnp.maximum(jnp.dot(h, w2_ref[...],
                            preferred_element_type=jnp.float32) + b2_ref[...],
                    0.0)
    o_ref[...] = jnp.sum(h * w3_ref[...], axis=1) + b3_ref[0, 0]


def _mlp(ue, me, w1a, w1b, b1, w2, b2, w3r, b3r):
    grid = (BATCH // BLK,)
    row_spec = pl.BlockSpec((BLK, D), lambda i: (i, 0))
    full = lambda shape: pl.BlockSpec(shape, lambda i: (0,) * len(shape))
    return pl.pallas_call(
        _mlp_body,
        grid=grid,
        in_specs=[
            row_spec, row_spec,
            full((D, 64)), full((D, 64)), full((1, 64)),
            full((64, 32)), full((1, 32)),
            full((1, 32)), full((1, 1)),
        ],
        out_specs=pl.BlockSpec((BLK,), lambda i: (i,)),
        out_shape=jax.ShapeDtypeStruct((BATCH,), jnp.float32),
    )(ue, me, w1a, w1b, b1, w2, b2, w3r, b3r)


def kernel(user, movie, user_table, movie_table, W1, b1, W2, b2, W3, b3):
    user = user.astype(jnp.int32)
    movie = movie.astype(jnp.int32)
    utab2 = user_table.reshape(-1, 8, 2 * D)
    mtab2 = movie_table.reshape(-1, 8, 2 * D)
    ue2, me2 = _make_gather()(user, movie, utab2, mtab2)
    ue = ue2.reshape(BATCH, D)
    me = me2.reshape(BATCH, D)
    return _mlp(ue, me,
                W1[:D], W1[D:], b1.reshape(1, 64),
                W2, b2.reshape(1, 32),
                W3.reshape(1, 32), b3.reshape(1, 1))


# speedup vs baseline: 2.3666x; 2.3666x over previous
import functools

import jax
import jax.numpy as jnp
from jax import lax
from jax.experimental import pallas as pl
from jax.experimental.pallas import tpu as pltpu
from jax.experimental.pallas import tpu_sc as plsc

BATCH = 16384
D = 64
NC, NS = 2, 16
NW = NC * NS
BPW = BATCH // NW
CHUNK = 256
NCHUNK = BPW // CHUNK
L = 16


def _gather_table(tab3, idx_v, out_hbm, base, rowbufs, sems):
    def issue(cc, b):
        def it(t, _):
            rvec = idx_v[pl.ds(cc * CHUNK + t * L, L)]
            gvec = lax.shift_right_logical(rvec, 3)
            svec = jnp.bitwise_and(rvec, 7)
            for lane in range(L):
                g = gvec[lane]
                s = svec[lane]
                pltpu.make_async_copy(
                    tab3.at[g, pl.ds(s, 1), :],
                    rowbufs[b].at[pl.ds(t * L + lane, 1), :],
                    sems[b],
                ).start()
            return ()

        lax.fori_loop(0, CHUNK // L, it, ())

    def drain(b):
        pltpu.make_async_copy(
            out_hbm.at[pl.ds(0, CHUNK)], rowbufs[b], sems[b]
        ).wait()

    issue(0, 0)
    for cc in range(NCHUNK):
        b = cc % 2
        nxt = cc + 1
        if nxt < NCHUNK:
            issue(nxt, 1 - b)
        drain(b)
        pltpu.sync_copy(rowbufs[b],
                        out_hbm.at[pl.ds(base + cc * CHUNK, CHUNK)])


def _gather_body(uidx_hbm, midx_hbm, utab_hbm, mtab_hbm,
                 uout_hbm, mout_hbm,
                 uidx_v, midx_v, rowbuf0, rowbuf1, sem0, sem1):
    wid = lax.axis_index("s") * NC + lax.axis_index("c")
    base = wid * BPW
    pltpu.sync_copy(uidx_hbm.at[pl.ds(base, BPW)], uidx_v)
    pltpu.sync_copy(midx_hbm.at[pl.ds(base, BPW)], midx_v)
    rowbufs = (rowbuf0, rowbuf1)
    sems = (sem0, sem1)
    _gather_table(utab_hbm, uidx_v, uout_hbm, base, rowbufs, sems)
    _gather_table(mtab_hbm, midx_v, mout_hbm, base, rowbufs, sems)


@functools.cache
def _make_gather():
    return pl.kernel(
        _gather_body,
        out_type=(jax.ShapeDtypeStruct((BATCH, D), jnp.float32),
                  jax.ShapeDtypeStruct((BATCH, D), jnp.float32)),
        mesh=plsc.VectorSubcoreMesh(core_axis_name="c", subcore_axis_name="s",
                                    num_cores=NC, num_subcores=NS),
        compiler_params=pltpu.CompilerParams(
            needs_layout_passes=False,
            disable_bounds_checks=True,
            disable_semaphore_checks=True,
            skip_device_barrier=True,
        ),
        scratch_types=[
            pltpu.VMEM((BPW,), jnp.int32),
            pltpu.VMEM((BPW,), jnp.int32),
            pltpu.VMEM((CHUNK, D), jnp.float32),
            pltpu.VMEM((CHUNK, D), jnp.float32),
            pltpu.SemaphoreType.DMA,
            pltpu.SemaphoreType.DMA,
        ],
    )


BLK = 2048


def _mlp_body(ue_ref, me_ref, w1a_ref, w1b_ref, b1_ref, w2_ref, b2_ref,
              w3_ref, b3_ref, o_ref):
    h = jnp.dot(ue_ref[...], w1a_ref[...], preferred_element_type=jnp.float32)
    h = h + jnp.dot(me_ref[...], w1b_ref[...],
                    preferred_element_type=jnp.float32)
    h = jnp.maximum(h + b1_ref[...], 0.0)
    h = jnp.maximum(jnp.dot(h, w2_ref[...],
                            preferred_element_type=jnp.float32) + b2_ref[...],
                    0.0)
    o_ref[...] = jnp.sum(h * w3_ref[...], axis=1) + b3_ref[0, 0]


def _mlp(ue, me, w1a, w1b, b1, w2, b2, w3r, b3r):
    grid = (BATCH // BLK,)
    row_spec = pl.BlockSpec((BLK, D), lambda i: (i, 0))
    full = lambda shape: pl.BlockSpec(shape, lambda i: (0,) * len(shape))
    return pl.pallas_call(
        _mlp_body,
        grid=grid,
        in_specs=[
            row_spec, row_spec,
            full((D, 64)), full((D, 64)), full((1, 64)),
            full((64, 32)), full((1, 32)),
            full((1, 32)), full((1, 1)),
        ],
        out_specs=pl.BlockSpec((BLK,), lambda i: (i,)),
        out_shape=jax.ShapeDtypeStruct((BATCH,), jnp.float32),
    )(ue, me, w1a, w1b, b1, w2, b2, w3r, b3r)


def kernel(user, movie, user_table, movie_table, W1, b1, W2, b2, W3, b3):
    user = user.astype(jnp.int32)
    movie = movie.astype(jnp.int32)
    utab3 = user_table.reshape(-1, 8, D)
    mtab3 = movie_table.reshape(-1, 8, D)
    ue, me = _make_gather()(user, movie, utab3, mtab3)
    return _mlp(ue, me,
                W1[:D], W1[D:], b1.reshape(1, 64),
                W2, b2.reshape(1, 32),
                W3.reshape(1, 32), b3.reshape(1, 1))
